# Initial kernel scaffold; baseline (speedup 1.0000x reference)
import jax
import jax.numpy as jnp
from jax.experimental import pallas as pl


def kernel(h, edge_index, W, b):
    raise NotImplementedError("write your pallas kernel here")


# speedup vs baseline: 6.8733x; 6.8733x over previous
import functools

import jax
import jax.numpy as jnp
from jax import lax
from jax.experimental import pallas as pl
from jax.experimental.pallas import tpu as pltpu
from jax.experimental.pallas import tpu_sc as plsc

N_NODES = 10000
N_EDGES = 320000
D_IN = 128
D_OUT = 128

NC = 2
NS = 16
D_HALF = D_IN // NC
E_PER_T = N_EDGES // NS
CHUNK = 125
NCHUNK = E_PER_T // CHUNK
N_PAD = 10240
ROWS_PER_TILE = N_PAD // NS
DEG_W = 16


@functools.partial(
    pl.kernel,
    out_type=(
        jax.ShapeDtypeStruct((NC, N_PAD, D_HALF), jnp.float32),
        jax.ShapeDtypeStruct((N_PAD, DEG_W), jnp.float32),
    ),
    mesh=plsc.VectorSubcoreMesh(core_axis_name="c", subcore_axis_name="s"),
    compiler_params=pltpu.CompilerParams(use_tc_tiling_on_sc=False),
    scratch_types=[
        pltpu.VMEM((NCHUNK, CHUNK), jnp.int32),
        pltpu.VMEM((NCHUNK, CHUNK), jnp.int32),
        pltpu.VMEM((CHUNK, D_HALF), jnp.float32),
        pltpu.VMEM((CHUNK, DEG_W), jnp.float32),
        pltpu.VMEM_SHARED((N_PAD, D_HALF), jnp.float32),
        pltpu.VMEM_SHARED((N_PAD, DEG_W), jnp.float32),
        pltpu.SemaphoreType.DMA,
    ],
)
def _sc_aggregate(hp_hbm, src_hbm, dst_hbm, ones_hbm, zacc_hbm, zdeg_hbm,
                  acc_out, deg_out,
                  idx_s, idx_d, rows, ones_v, acc_sh, deg_sh, sem):
    cid = lax.axis_index("c")
    sid = lax.axis_index("s")

    pltpu.sync_copy(src_hbm.at[sid], idx_s)
    pltpu.sync_copy(dst_hbm.at[sid], idx_d)
    pltpu.sync_copy(ones_hbm, ones_v)

    base = sid * ROWS_PER_TILE
    pltpu.sync_copy(zacc_hbm, acc_sh.at[pl.ds(base, ROWS_PER_TILE)])
    pltpu.sync_copy(zdeg_hbm, deg_sh.at[pl.ds(base, ROWS_PER_TILE)])
    plsc.subcore_barrier()

    def body(c, carry):
        pltpu.async_copy(hp_hbm.at[cid].at[idx_s.at[c]], rows, sem).wait()
        pltpu.sync_copy(rows, acc_sh.at[idx_d.at[c]], add=True)

        @pl.when(cid == 0)
        def _():
            pltpu.sync_copy(ones_v, deg_sh.at[idx_d.at[c]], add=True)

        return carry

    lax.fori_loop(0, NCHUNK, body, 0)
    plsc.subcore_barrier()

    pltpu.sync_copy(acc_sh.at[pl.ds(base, ROWS_PER_TILE)],
                    acc_out.at[cid, pl.ds(base, ROWS_PER_TILE)])

    @pl.when(cid == 0)
    def _():
        pltpu.sync_copy(deg_sh.at[pl.ds(base, ROWS_PER_TILE)],
                        deg_out.at[pl.ds(base, ROWS_PER_TILE)])


ROW_BLK = 1000


def _tc_linear_body(h_ref, a0_ref, a1_ref, d_ref,
                    w1_ref, w2a_ref, w2b_ref, b_ref, o_ref):
    rdeg = 1.0 / jnp.maximum(d_ref[:, 0:1], 1.0)
    o_ref[...] = (
        jnp.dot(h_ref[...], w1_ref[...], preferred_element_type=jnp.float32)
        + jnp.dot(a0_ref[...] * rdeg, w2a_ref[...],
                  preferred_element_type=jnp.float32)
        + jnp.dot(a1_ref[...] * rdeg, w2b_ref[...],
                  preferred_element_type=jnp.float32)
        + b_ref[...]
    )


def _tc_linear(h, acc0, acc1, deg, w1t, w2ta, w2tb, b2d):
    grid = (N_NODES // ROW_BLK,)
    return pl.pallas_call(
        _tc_linear_body,
        grid=grid,
        in_specs=[
            pl.BlockSpec((ROW_BLK, D_IN), lambda i: (i, 0)),
            pl.BlockSpec((ROW_BLK, D_HALF), lambda i: (i, 0)),
            pl.BlockSpec((ROW_BLK, D_HALF), lambda i: (i, 0)),
            pl.BlockSpec((ROW_BLK, DEG_W), lambda i: (i, 0)),
            pl.BlockSpec((D_IN, D_OUT), lambda i: (0, 0)),
            pl.BlockSpec((D_HALF, D_OUT), lambda i: (0, 0)),
            pl.BlockSpec((D_HALF, D_OUT), lambda i: (0, 0)),
            pl.BlockSpec((1, D_OUT), lambda i: (0, 0)),
        ],
        out_specs=pl.BlockSpec((ROW_BLK, D_OUT), lambda i: (i, 0)),
        out_shape=jax.ShapeDtypeStruct((N_NODES, D_OUT), jnp.float32),
    )(h, acc0, acc1, deg, w1t, w2ta, w2tb, b2d)


def kernel(h, edge_index, W, b):
    src = edge_index[0].astype(jnp.int32).reshape(NS, NCHUNK, CHUNK)
    dst = edge_index[1].astype(jnp.int32).reshape(NS, NCHUNK, CHUNK)
    h_pair = jnp.stack([h[:, :D_HALF], h[:, D_HALF:]])
    ones = jnp.ones((CHUNK, DEG_W), dtype=jnp.float32)
    zacc = jnp.zeros((ROWS_PER_TILE, D_HALF), dtype=jnp.float32)
    zdeg = jnp.zeros((ROWS_PER_TILE, DEG_W), dtype=jnp.float32)

    acc, deg = _sc_aggregate(h_pair, src, dst, ones, zacc, zdeg)

    w1t = W[:, :D_IN].T
    w2ta = W[:, D_IN:D_IN + D_HALF].T
    w2tb = W[:, D_IN + D_HALF:].T
    b2d = b.reshape(1, D_OUT)
    return _tc_linear(h, acc[0], acc[1], deg, w1t, w2ta, w2tb, b2d)


# speedup vs baseline: 8.5788x; 1.2481x over previous
import functools

import jax
import jax.numpy as jnp
from jax import lax
from jax.experimental import pallas as pl
from jax.experimental.pallas import tpu as pltpu
from jax.experimental.pallas import tpu_sc as plsc

N_NODES = 10000
N_EDGES = 320000
D_IN = 128
D_OUT = 128

NC = 2
NS = 16
D_HALF = D_IN // NC
E_PER_T = N_EDGES // NS
CHUNK = 125
NCHUNK = E_PER_T // CHUNK
N_PAD = 10240
ROWS_PER_TILE = N_PAD // NS
DEG_W = 16


@functools.partial(
    pl.kernel,
    out_type=(
        jax.ShapeDtypeStruct((NC, N_PAD, D_HALF), jnp.float32),
        jax.ShapeDtypeStruct((NC, N_PAD, DEG_W), jnp.float32),
    ),
    mesh=plsc.VectorSubcoreMesh(core_axis_name="c", subcore_axis_name="s"),
    compiler_params=pltpu.CompilerParams(use_tc_tiling_on_sc=False),
    scratch_types=[
        pltpu.VMEM((NCHUNK, CHUNK), jnp.int32),
        pltpu.VMEM((NCHUNK, CHUNK), jnp.int32),
        pltpu.VMEM((2, CHUNK, D_HALF), jnp.float32),
        pltpu.VMEM((CHUNK, DEG_W), jnp.float32),
        pltpu.VMEM_SHARED((N_PAD, D_HALF), jnp.float32),
        pltpu.VMEM_SHARED((N_PAD, DEG_W), jnp.float32),
        pltpu.SemaphoreType.DMA,
        pltpu.SemaphoreType.DMA,
    ],
)
def _sc_aggregate(hp_hbm, src_hbm, dst_hbm, ones_hbm, zacc_hbm, zdeg_hbm,
                  acc_out, deg_out,
                  idx_s, idx_d, rows, ones_v, acc_sh, deg_sh, sem0, sem1):
    cid = lax.axis_index("c")
    sid = lax.axis_index("s")
    sems = (sem0, sem1)

    pltpu.sync_copy(src_hbm.at[sid], idx_s)
    pltpu.sync_copy(dst_hbm.at[sid], idx_d)
    pltpu.sync_copy(ones_hbm, ones_v)

    base = sid * ROWS_PER_TILE
    pltpu.sync_copy(zacc_hbm, acc_sh.at[pl.ds(base, ROWS_PER_TILE)])
    pltpu.sync_copy(zdeg_hbm, deg_sh.at[pl.ds(base, ROWS_PER_TILE)])
    plsc.subcore_barrier()

    def start_gather(c, b):
        pltpu.async_copy(hp_hbm.at[cid].at[idx_s.at[c]], rows.at[b], sems[b])

    def wait_gather(c, b):
        pltpu.make_async_copy(hp_hbm.at[cid].at[idx_s.at[c]],
                              rows.at[b], sems[b]).wait()

    start_gather(0, 0)

    def body(g, carry):
        for b in range(2):
            c = 2 * g + b
            wait_gather(c, b)

            @pl.when(c < NCHUNK - 1)
            def _():
                start_gather(c + 1, 1 - b)

            pltpu.sync_copy(rows.at[b], acc_sh.at[idx_d.at[c]], add=True)

            @pl.when(c // (NCHUNK // 2) == cid)
            def _():
                pltpu.sync_copy(ones_v, deg_sh.at[idx_d.at[c]], add=True)

        return carry

    lax.fori_loop(0, NCHUNK // 2, body, 0)
    plsc.subcore_barrier()

    pltpu.sync_copy(acc_sh.at[pl.ds(base, ROWS_PER_TILE)],
                    acc_out.at[cid, pl.ds(base, ROWS_PER_TILE)])
    pltpu.sync_copy(deg_sh.at[pl.ds(base, ROWS_PER_TILE)],
                    deg_out.at[cid, pl.ds(base, ROWS_PER_TILE)])


ROW_BLK = 1000


def _tc_linear_body(h_ref, a0_ref, a1_ref, d0_ref, d1_ref,
                    w1_ref, w2a_ref, w2b_ref, b_ref, o_ref):
    deg = d0_ref[:, 0:1] + d1_ref[:, 0:1]
    rdeg = 1.0 / jnp.maximum(deg, 1.0)
    o_ref[...] = (
        jnp.dot(h_ref[...], w1_ref[...], preferred_element_type=jnp.float32)
        + jnp.dot(a0_ref[...] * rdeg, w2a_ref[...],
                  preferred_element_type=jnp.float32)
        + jnp.dot(a1_ref[...] * rdeg, w2b_ref[...],
                  preferred_element_type=jnp.float32)
        + b_ref[...]
    )


def _tc_linear(h, acc0, acc1, deg0, deg1, w1t, w2ta, w2tb, b2d):
    grid = (N_NODES // ROW_BLK,)
    return pl.pallas_call(
        _tc_linear_body,
        grid=grid,
        in_specs=[
            pl.BlockSpec((ROW_BLK, D_IN), lambda i: (i, 0)),
            pl.BlockSpec((ROW_BLK, D_HALF), lambda i: (i, 0)),
            pl.BlockSpec((ROW_BLK, D_HALF), lambda i: (i, 0)),
            pl.BlockSpec((ROW_BLK, DEG_W), lambda i: (i, 0)),
            pl.BlockSpec((ROW_BLK, DEG_W), lambda i: (i, 0)),
            pl.BlockSpec((D_IN, D_OUT), lambda i: (0, 0)),
            pl.BlockSpec((D_HALF, D_OUT), lambda i: (0, 0)),
            pl.BlockSpec((D_HALF, D_OUT), lambda i: (0, 0)),
            pl.BlockSpec((1, D_OUT), lambda i: (0, 0)),
        ],
        out_specs=pl.BlockSpec((ROW_BLK, D_OUT), lambda i: (i, 0)),
        out_shape=jax.ShapeDtypeStruct((N_NODES, D_OUT), jnp.float32),
    )(h, acc0, acc1, deg0, deg1, w1t, w2ta, w2tb, b2d)


def kernel(h, edge_index, W, b):
    src = edge_index[0].astype(jnp.int32).reshape(NS, NCHUNK, CHUNK)
    dst = edge_index[1].astype(jnp.int32).reshape(NS, NCHUNK, CHUNK)
    h_pair = jnp.stack([h[:, :D_HALF], h[:, D_HALF:]])
    ones = jnp.ones((CHUNK, DEG_W), dtype=jnp.float32)
    zacc = jnp.zeros((ROWS_PER_TILE, D_HALF), dtype=jnp.float32)
    zdeg = jnp.zeros((ROWS_PER_TILE, DEG_W), dtype=jnp.float32)

    acc, deg = _sc_aggregate(h_pair, src, dst, ones, zacc, zdeg)

    w1t = W[:, :D_IN].T
    w2ta = W[:, D_IN:D_IN + D_HALF].T
    w2tb = W[:, D_IN + D_HALF:].T
    b2d = b.reshape(1, D_OUT)
    return _tc_linear(h, acc[0], acc[1], deg[0], deg[1], w1t, w2ta, w2tb, b2d)


# speedup vs baseline: 10.6224x; 1.2382x over previous
import functools

import jax
import jax.numpy as jnp
from jax import lax
from jax.experimental import pallas as pl
from jax.experimental.pallas import tpu as pltpu
from jax.experimental.pallas import tpu_sc as plsc

N_NODES = 10000
N_EDGES = 320000
D_IN = 128
D_OUT = 128

NC = 2
NS = 16
D_HALF = D_IN // NC
E_PER_T = N_EDGES // NS
CHUNK = 125
NCHUNK = E_PER_T // CHUNK
N_PAD = 10240
ROWS_PER_TILE = N_PAD // NS
DEG_W = 16


@functools.partial(
    pl.kernel,
    out_type=(
        jax.ShapeDtypeStruct((NC, N_PAD, D_HALF), jnp.float32),
        jax.ShapeDtypeStruct((NC, N_PAD, DEG_W), jnp.float32),
    ),
    mesh=plsc.VectorSubcoreMesh(core_axis_name="c", subcore_axis_name="s"),
    compiler_params=pltpu.CompilerParams(use_tc_tiling_on_sc=False),
    scratch_types=[
        pltpu.VMEM((NCHUNK, CHUNK), jnp.int32),
        pltpu.VMEM((NCHUNK, CHUNK), jnp.int32),
        pltpu.VMEM((4, CHUNK, D_HALF), jnp.float32),
        pltpu.VMEM((CHUNK, DEG_W), jnp.float32),
        pltpu.VMEM_SHARED((N_PAD, D_HALF), jnp.float32),
        pltpu.VMEM_SHARED((N_PAD, DEG_W), jnp.float32),
        [pltpu.SemaphoreType.DMA] * 4,
        [pltpu.SemaphoreType.DMA] * 4,
    ],
)
def _sc_aggregate(hp_hbm, src_hbm, dst_hbm, ones_hbm, zacc_hbm, zdeg_hbm,
                  acc_out, deg_out,
                  idx_s, idx_d, rows, ones_v, acc_sh, deg_sh, gsems, ssems):
    cid = lax.axis_index("c")
    sid = lax.axis_index("s")

    pltpu.sync_copy(src_hbm.at[sid], idx_s)
    pltpu.sync_copy(dst_hbm.at[sid], idx_d)
    pltpu.sync_copy(ones_hbm, ones_v)

    base = sid * ROWS_PER_TILE
    pltpu.sync_copy(zacc_hbm, acc_sh.at[pl.ds(base, ROWS_PER_TILE)])
    pltpu.sync_copy(zdeg_hbm, deg_sh.at[pl.ds(base, ROWS_PER_TILE)])
    plsc.subcore_barrier()

    def start_gather(c, b):
        pltpu.async_copy(hp_hbm.at[cid].at[idx_s.at[c]], rows.at[b], gsems[b])

    def wait_gather(c, b):
        pltpu.make_async_copy(hp_hbm.at[cid].at[idx_s.at[c]],
                              rows.at[b], gsems[b]).wait()

    def start_scatter(c, b):
        pltpu.async_copy(rows.at[b], acc_sh.at[idx_d.at[c]], ssems[b],
                         add=True)

    def wait_scatter(c, b):
        pltpu.make_async_copy(rows.at[b], acc_sh.at[idx_d.at[c]],
                              ssems[b]).wait()

    start_gather(0, 0)
    start_gather(1, 1)

    def body(q, carry):
        for bb in range(4):
            c = 4 * q + bb
            wait_gather(c, bb)
            start_scatter(c, bb)

            @pl.when(c // (NCHUNK // 2) == cid)
            def _():
                pltpu.sync_copy(ones_v, deg_sh.at[idx_d.at[c]], add=True)

            nb = (bb + 2) % 4

            @pl.when(c + 2 < NCHUNK)
            def _():
                @pl.when(c >= 2)
                def _():
                    wait_scatter(c - 2, nb)

                start_gather(c + 2, nb)

        return carry

    lax.fori_loop(0, NCHUNK // 4, body, 0)
    for bb in range(4):
        wait_scatter(NCHUNK - 4 + bb, bb)
    plsc.subcore_barrier()

    pltpu.sync_copy(acc_sh.at[pl.ds(base, ROWS_PER_TILE)],
                    acc_out.at[cid, pl.ds(base, ROWS_PER_TILE)])
    pltpu.sync_copy(deg_sh.at[pl.ds(base, ROWS_PER_TILE)],
                    deg_out.at[cid, pl.ds(base, ROWS_PER_TILE)])


ROW_BLK = 1000


def _tc_linear_body(h_ref, a0_ref, a1_ref, d0_ref, d1_ref,
                    w1_ref, w2a_ref, w2b_ref, b_ref, o_ref):
    deg = d0_ref[:, 0:1] + d1_ref[:, 0:1]
    rdeg = 1.0 / jnp.maximum(deg, 1.0)
    o_ref[...] = (
        jnp.dot(h_ref[...], w1_ref[...], preferred_element_type=jnp.float32)
        + jnp.dot(a0_ref[...] * rdeg, w2a_ref[...],
                  preferred_element_type=jnp.float32)
        + jnp.dot(a1_ref[...] * rdeg, w2b_ref[...],
                  preferred_element_type=jnp.float32)
        + b_ref[...]
    )


def _tc_linear(h, acc0, acc1, deg0, deg1, w1t, w2ta, w2tb, b2d):
    grid = (N_NODES // ROW_BLK,)
    return pl.pallas_call(
        _tc_linear_body,
        grid=grid,
        in_specs=[
            pl.BlockSpec((ROW_BLK, D_IN), lambda i: (i, 0)),
            pl.BlockSpec((ROW_BLK, D_HALF), lambda i: (i, 0)),
            pl.BlockSpec((ROW_BLK, D_HALF), lambda i: (i, 0)),
            pl.BlockSpec((ROW_BLK, DEG_W), lambda i: (i, 0)),
            pl.BlockSpec((ROW_BLK, DEG_W), lambda i: (i, 0)),
            pl.BlockSpec((D_IN, D_OUT), lambda i: (0, 0)),
            pl.BlockSpec((D_HALF, D_OUT), lambda i: (0, 0)),
            pl.BlockSpec((D_HALF, D_OUT), lambda i: (0, 0)),
            pl.BlockSpec((1, D_OUT), lambda i: (0, 0)),
        ],
        out_specs=pl.BlockSpec((ROW_BLK, D_OUT), lambda i: (i, 0)),
        out_shape=jax.ShapeDtypeStruct((N_NODES, D_OUT), jnp.float32),
    )(h, acc0, acc1, deg0, deg1, w1t, w2ta, w2tb, b2d)


def kernel(h, edge_index, W, b):
    src = edge_index[0].astype(jnp.int32).reshape(NS, NCHUNK, CHUNK)
    dst = edge_index[1].astype(jnp.int32).reshape(NS, NCHUNK, CHUNK)
    h_pair = jnp.stack([h[:, :D_HALF], h[:, D_HALF:]])
    ones = jnp.ones((CHUNK, DEG_W), dtype=jnp.float32)
    zacc = jnp.zeros((ROWS_PER_TILE, D_HALF), dtype=jnp.float32)
    zdeg = jnp.zeros((ROWS_PER_TILE, DEG_W), dtype=jnp.float32)

    acc, deg = _sc_aggregate(h_pair, src, dst, ones, zacc, zdeg)

    w1t = W[:, :D_IN].T
    w2ta = W[:, D_IN:D_IN + D_HALF].T
    w2tb = W[:, D_IN + D_HALF:].T
    b2d = b.reshape(1, D_OUT)
    return _tc_linear(h, acc[0], acc[1], deg[0], deg[1], w1t, w2ta, w2tb, b2d)


# speedup vs baseline: 12.2393x; 1.1522x over previous
import functools

import jax
import jax.numpy as jnp
from jax import lax
from jax.experimental import pallas as pl
from jax.experimental.pallas import tpu as pltpu
from jax.experimental.pallas import tpu_sc as plsc

N_NODES = 10000
N_EDGES = 320000
D_IN = 128
D_OUT = 128

NC = 2
NS = 16
D_HALF = D_IN // NC
E_PER_T = N_EDGES // NS
CHUNK = 125
NCHUNK = E_PER_T // CHUNK
N_PAD = 10240
ROWS_PER_TILE = N_PAD // NS
DEG_W = 16


@functools.partial(
    pl.kernel,
    out_type=(
        jax.ShapeDtypeStruct((NC, N_PAD, D_HALF), jnp.float32),
        jax.ShapeDtypeStruct((NC, N_PAD, DEG_W), jnp.float32),
    ),
    mesh=plsc.VectorSubcoreMesh(core_axis_name="c", subcore_axis_name="s"),
    compiler_params=pltpu.CompilerParams(use_tc_tiling_on_sc=False),
    scratch_types=[
        pltpu.VMEM((NCHUNK, CHUNK), jnp.int32),
        pltpu.VMEM((NCHUNK, CHUNK), jnp.int32),
        pltpu.VMEM((4, CHUNK, D_HALF), jnp.float32),
        pltpu.VMEM((CHUNK, DEG_W), jnp.float32),
        pltpu.VMEM_SHARED((N_PAD, D_HALF), jnp.float32),
        pltpu.VMEM_SHARED((N_PAD, DEG_W), jnp.float32),
        [pltpu.SemaphoreType.DMA] * 4,
        [pltpu.SemaphoreType.DMA] * 4,
    ],
)
def _sc_aggregate(h2_hbm, src_hbm, dst_hbm, ones_hbm, zacc_hbm, zdeg_hbm,
                  acc_out, deg_out,
                  idx_s, idx_d, rows, ones_v, acc_sh, deg_sh, gsems, ssems):
    cid = lax.axis_index("c")
    sid = lax.axis_index("s")

    pltpu.sync_copy(src_hbm.at[sid], idx_s)
    pltpu.sync_copy(dst_hbm.at[sid], idx_d)
    pltpu.sync_copy(ones_hbm, ones_v)

    base = sid * ROWS_PER_TILE
    pltpu.sync_copy(zacc_hbm, acc_sh.at[pl.ds(base, ROWS_PER_TILE)])
    pltpu.sync_copy(zdeg_hbm, deg_sh.at[pl.ds(base, ROWS_PER_TILE)])
    plsc.subcore_barrier()

    h_view = h2_hbm.at[pl.ds(cid, 2 * N_NODES - 1)]

    def start_gather(c, b):
        pltpu.async_copy(h_view.at[idx_s.at[c]], rows.at[b], gsems[b])

    def wait_gather(c, b):
        pltpu.make_async_copy(h_view.at[idx_s.at[c]],
                              rows.at[b], gsems[b]).wait()

    def start_scatter(c, b):
        pltpu.async_copy(rows.at[b], acc_sh.at[idx_d.at[c]], ssems[b],
                         add=True)

    def wait_scatter(c, b):
        pltpu.make_async_copy(rows.at[b], acc_sh.at[idx_d.at[c]],
                              ssems[b]).wait()

    start_gather(0, 0)
    start_gather(1, 1)

    def body(q, carry):
        for bb in range(4):
            c = 4 * q + bb
            wait_gather(c, bb)
            start_scatter(c, bb)

            @pl.when(c // (NCHUNK // 2) == cid)
            def _():
                pltpu.sync_copy(ones_v, deg_sh.at[idx_d.at[c]], add=True)

            nb = (bb + 2) % 4

            @pl.when(c + 2 < NCHUNK)
            def _():
                @pl.when(c >= 2)
                def _():
                    wait_scatter(c - 2, nb)

                start_gather(c + 2, nb)

        return carry

    lax.fori_loop(0, NCHUNK // 4, body, 0)
    for bb in range(4):
        wait_scatter(NCHUNK - 4 + bb, bb)
    plsc.subcore_barrier()

    pltpu.sync_copy(acc_sh.at[pl.ds(base, ROWS_PER_TILE)],
                    acc_out.at[cid, pl.ds(base, ROWS_PER_TILE)])
    pltpu.sync_copy(deg_sh.at[pl.ds(base, ROWS_PER_TILE)],
                    deg_out.at[cid, pl.ds(base, ROWS_PER_TILE)])


ROW_BLK = 1000


def _tc_linear_body(h_ref, a0_ref, a1_ref, d0_ref, d1_ref,
                    w1_ref, w2a_ref, w2b_ref, b_ref, o_ref):
    deg = d0_ref[0, :, 0:1] + d1_ref[0, :, 0:1]
    rdeg = 1.0 / jnp.maximum(deg, 1.0)
    o_ref[...] = (
        jnp.dot(h_ref[...], w1_ref[...], preferred_element_type=jnp.float32)
        + jnp.dot(a0_ref[0] * rdeg, w2a_ref[...],
                  preferred_element_type=jnp.float32)
        + jnp.dot(a1_ref[0] * rdeg, w2b_ref[...],
                  preferred_element_type=jnp.float32)
        + b_ref[...]
    )


def _tc_linear(h, acc, deg, w1t, w2ta, w2tb, b2d):
    grid = (N_NODES // ROW_BLK,)
    return pl.pallas_call(
        _tc_linear_body,
        grid=grid,
        in_specs=[
            pl.BlockSpec((ROW_BLK, D_IN), lambda i: (i, 0)),
            pl.BlockSpec((1, ROW_BLK, D_HALF), lambda i: (0, i, 0)),
            pl.BlockSpec((1, ROW_BLK, D_HALF), lambda i: (1, i, 0)),
            pl.BlockSpec((1, ROW_BLK, DEG_W), lambda i: (0, i, 0)),
            pl.BlockSpec((1, ROW_BLK, DEG_W), lambda i: (1, i, 0)),
            pl.BlockSpec((D_IN, D_OUT), lambda i: (0, 0)),
            pl.BlockSpec((D_HALF, D_OUT), lambda i: (0, 0)),
            pl.BlockSpec((D_HALF, D_OUT), lambda i: (0, 0)),
            pl.BlockSpec((1, D_OUT), lambda i: (0, 0)),
        ],
        out_specs=pl.BlockSpec((ROW_BLK, D_OUT), lambda i: (i, 0)),
        out_shape=jax.ShapeDtypeStruct((N_NODES, D_OUT), jnp.float32),
    )(h, acc, acc, deg, deg, w1t, w2ta, w2tb, b2d)


def kernel(h, edge_index, W, b):
    src2 = (edge_index[0].astype(jnp.int32) * 2).reshape(NS, NCHUNK, CHUNK)
    dst = edge_index[1].astype(jnp.int32).reshape(NS, NCHUNK, CHUNK)
    h2 = h.reshape(2 * N_NODES, D_HALF)
    ones = jnp.ones((CHUNK, DEG_W), dtype=jnp.float32)
    zacc = jnp.zeros((ROWS_PER_TILE, D_HALF), dtype=jnp.float32)
    zdeg = jnp.zeros((ROWS_PER_TILE, DEG_W), dtype=jnp.float32)

    acc, deg = _sc_aggregate(h2, src2, dst, ones, zacc, zdeg)

    w1t = W[:, :D_IN].T
    w2ta = W[:, D_IN:D_IN + D_HALF].T
    w2tb = W[:, D_IN + D_HALF:].T
    b2d = b.reshape(1, D_OUT)
    return _tc_linear(h, acc, deg, w1t, w2ta, w2tb, b2d)


# speedup vs baseline: 13.6683x; 1.1168x over previous
import functools

import jax
import jax.numpy as jnp
from jax import lax
from jax.experimental import pallas as pl
from jax.experimental.pallas import tpu as pltpu
from jax.experimental.pallas import tpu_sc as plsc

N_NODES = 10000
N_EDGES = 320000
D_IN = 128
D_OUT = 128

NC = 2
NS = 16
D_HALF = D_IN // NC
E_PER_T = N_EDGES // NS
CHUNK = 125
NCHUNK = E_PER_T // CHUNK
N_PAD = 10240
ROWS_PER_TILE = N_PAD // NS
DEG_W = 16


@functools.partial(
    pl.kernel,
    out_type=(
        jax.ShapeDtypeStruct((NC, N_PAD, D_HALF), jnp.bfloat16),
        jax.ShapeDtypeStruct((NC, N_PAD, DEG_W), jnp.float32),
    ),
    mesh=plsc.VectorSubcoreMesh(core_axis_name="c", subcore_axis_name="s"),
    compiler_params=pltpu.CompilerParams(use_tc_tiling_on_sc=False),
    scratch_types=[
        pltpu.VMEM((NCHUNK, CHUNK), jnp.int32),
        pltpu.VMEM((NCHUNK, CHUNK), jnp.int32),
        pltpu.VMEM((4, CHUNK, D_HALF), jnp.bfloat16),
        pltpu.VMEM((CHUNK, DEG_W), jnp.float32),
        pltpu.VMEM_SHARED((N_PAD, D_HALF), jnp.bfloat16),
        pltpu.VMEM_SHARED((N_PAD, DEG_W), jnp.float32),
        [pltpu.SemaphoreType.DMA] * 4,
        [pltpu.SemaphoreType.DMA] * 4,
    ],
)
def _sc_aggregate(h2_hbm, src_hbm, dst_hbm, ones_hbm, zacc_hbm, zdeg_hbm,
                  acc_out, deg_out,
                  idx_s, idx_d, rows, ones_v, acc_sh, deg_sh, gsems, ssems):
    cid = lax.axis_index("c")
    sid = lax.axis_index("s")

    pltpu.sync_copy(src_hbm.at[sid], idx_s)
    pltpu.sync_copy(dst_hbm.at[sid], idx_d)
    pltpu.sync_copy(ones_hbm, ones_v)

    base = sid * ROWS_PER_TILE
    pltpu.sync_copy(zacc_hbm, acc_sh.at[pl.ds(base, ROWS_PER_TILE)])
    pltpu.sync_copy(zdeg_hbm, deg_sh.at[pl.ds(base, ROWS_PER_TILE)])
    plsc.subcore_barrier()

    h_view = h2_hbm.at[pl.ds(cid, 2 * N_NODES - 1)]

    def start_gather(c, b):
        pltpu.async_copy(h_view.at[idx_s.at[c]], rows.at[b], gsems[b])

    def wait_gather(c, b):
        pltpu.make_async_copy(h_view.at[idx_s.at[c]],
                              rows.at[b], gsems[b]).wait()

    def start_scatter(c, b):
        pltpu.async_copy(rows.at[b], acc_sh.at[idx_d.at[c]], ssems[b],
                         add=True)

    def wait_scatter(c, b):
        pltpu.make_async_copy(rows.at[b], acc_sh.at[idx_d.at[c]],
                              ssems[b]).wait()

    start_gather(0, 0)
    start_gather(1, 1)

    def body(q, carry):
        for bb in range(4):
            c = 4 * q + bb
            wait_gather(c, bb)
            start_scatter(c, bb)

            @pl.when(c // (NCHUNK // 2) == cid)
            def _():
                pltpu.sync_copy(ones_v, deg_sh.at[idx_d.at[c]], add=True)

            nb = (bb + 2) % 4

            @pl.when(c + 2 < NCHUNK)
            def _():
                @pl.when(c >= 2)
                def _():
                    wait_scatter(c - 2, nb)

                start_gather(c + 2, nb)

        return carry

    lax.fori_loop(0, NCHUNK // 4, body, 0)
    for bb in range(4):
        wait_scatter(NCHUNK - 4 + bb, bb)
    plsc.subcore_barrier()

    pltpu.sync_copy(acc_sh.at[pl.ds(base, ROWS_PER_TILE)],
                    acc_out.at[cid, pl.ds(base, ROWS_PER_TILE)])
    pltpu.sync_copy(deg_sh.at[pl.ds(base, ROWS_PER_TILE)],
                    deg_out.at[cid, pl.ds(base, ROWS_PER_TILE)])


ROW_BLK = 1000


def _tc_linear_body(h_ref, a0_ref, a1_ref, d0_ref, d1_ref,
                    w1_ref, w2a_ref, w2b_ref, b_ref, o_ref):
    deg = d0_ref[0, :, 0:1] + d1_ref[0, :, 0:1]
    rdeg = 1.0 / jnp.maximum(deg, 1.0)
    o_ref[...] = (
        jnp.dot(h_ref[...], w1_ref[...], preferred_element_type=jnp.float32)
        + jnp.dot(a0_ref[0] * rdeg, w2a_ref[...],
                  preferred_element_type=jnp.float32)
        + jnp.dot(a1_ref[0] * rdeg, w2b_ref[...],
                  preferred_element_type=jnp.float32)
        + b_ref[...]
    )


def _tc_linear(h, acc, deg, w1t, w2ta, w2tb, b2d):
    grid = (N_NODES // ROW_BLK,)
    return pl.pallas_call(
        _tc_linear_body,
        grid=grid,
        in_specs=[
            pl.BlockSpec((ROW_BLK, D_IN), lambda i: (i, 0)),
            pl.BlockSpec((1, ROW_BLK, D_HALF), lambda i: (0, i, 0)),
            pl.BlockSpec((1, ROW_BLK, D_HALF), lambda i: (1, i, 0)),
            pl.BlockSpec((1, ROW_BLK, DEG_W), lambda i: (0, i, 0)),
            pl.BlockSpec((1, ROW_BLK, DEG_W), lambda i: (1, i, 0)),
            pl.BlockSpec((D_IN, D_OUT), lambda i: (0, 0)),
            pl.BlockSpec((D_HALF, D_OUT), lambda i: (0, 0)),
            pl.BlockSpec((D_HALF, D_OUT), lambda i: (0, 0)),
            pl.BlockSpec((1, D_OUT), lambda i: (0, 0)),
        ],
        out_specs=pl.BlockSpec((ROW_BLK, D_OUT), lambda i: (i, 0)),
        out_shape=jax.ShapeDtypeStruct((N_NODES, D_OUT), jnp.float32),
    )(h, acc, acc, deg, deg, w1t, w2ta, w2tb, b2d)


def kernel(h, edge_index, W, b):
    src2 = (edge_index[0].astype(jnp.int32) * 2).reshape(NS, NCHUNK, CHUNK)
    dst = edge_index[1].astype(jnp.int32).reshape(NS, NCHUNK, CHUNK)
    h2 = h.astype(jnp.bfloat16).reshape(2 * N_NODES, D_HALF)
    ones = jnp.ones((CHUNK, DEG_W), dtype=jnp.float32)
    zacc = jnp.zeros((ROWS_PER_TILE, D_HALF), dtype=jnp.bfloat16)
    zdeg = jnp.zeros((ROWS_PER_TILE, DEG_W), dtype=jnp.float32)

    acc, deg = _sc_aggregate(h2, src2, dst, ones, zacc, zdeg)

    w1t = W[:, :D_IN].T
    w2ta = W[:, D_IN:D_IN + D_HALF].T
    w2tb = W[:, D_IN + D_HALF:].T
    b2d = b.reshape(1, D_OUT)
    return _tc_linear(h, acc, deg, w1t, w2ta, w2tb, b2d)


# speedup vs baseline: 15.8447x; 1.1592x over previous
import functools

import jax
import jax.numpy as jnp
from jax import lax
from jax.experimental import pallas as pl
from jax.experimental.pallas import tpu as pltpu
from jax.experimental.pallas import tpu_sc as plsc

N_NODES = 10000
N_EDGES = 320000
D_IN = 128
D_OUT = 128

NC = 2
NS = 16
D_HALF = D_IN // NC
E_PER_T = N_EDGES // NS
CHUNK = 125
NCHUNK = E_PER_T // CHUNK
N_PAD = 10240
ROWS_PER_TILE = N_PAD // NS
DEG_W = 16


@functools.partial(
    pl.kernel,
    out_type=(
        jax.ShapeDtypeStruct((NC, N_PAD, D_HALF), jnp.bfloat16),
        jax.ShapeDtypeStruct((NC, N_PAD, DEG_W), jnp.float32),
    ),
    mesh=plsc.VectorSubcoreMesh(core_axis_name="c", subcore_axis_name="s"),
    compiler_params=pltpu.CompilerParams(use_tc_tiling_on_sc=False),
    scratch_types=[
        pltpu.VMEM((NCHUNK, CHUNK), jnp.int32),
        pltpu.VMEM((NCHUNK, CHUNK), jnp.int32),
        pltpu.VMEM((8, CHUNK, D_HALF), jnp.bfloat16),
        pltpu.VMEM((CHUNK, DEG_W), jnp.float32),
        pltpu.VMEM_SHARED((N_PAD, D_HALF), jnp.bfloat16),
        pltpu.VMEM_SHARED((N_PAD, DEG_W), jnp.float32),
        [pltpu.SemaphoreType.DMA] * 8,
        [pltpu.SemaphoreType.DMA] * 8,
        pltpu.SemaphoreType.DMA,
    ],
)
def _sc_aggregate(h2_hbm, src_hbm, dst_hbm, ones_hbm, zacc_hbm, zdeg_hbm,
                  acc_out, deg_out,
                  idx_s, idx_d, rows, ones_v, acc_sh, deg_sh, gsems, ssems, dsem):
    cid = lax.axis_index("c")
    sid = lax.axis_index("s")

    pltpu.sync_copy(src_hbm.at[sid], idx_s)
    pltpu.sync_copy(dst_hbm.at[sid], idx_d)
    pltpu.sync_copy(ones_hbm, ones_v)

    base = sid * ROWS_PER_TILE
    pltpu.sync_copy(zacc_hbm, acc_sh.at[pl.ds(base, ROWS_PER_TILE)])
    pltpu.sync_copy(zdeg_hbm, deg_sh.at[pl.ds(base, ROWS_PER_TILE)])
    plsc.subcore_barrier()

    h_view = h2_hbm.at[pl.ds(cid, 2 * N_NODES - 1)]

    def start_gather(c, b):
        pltpu.async_copy(h_view.at[idx_s.at[c]], rows.at[b], gsems[b])

    def wait_gather(c, b):
        pltpu.make_async_copy(h_view.at[idx_s.at[c]],
                              rows.at[b], gsems[b]).wait()

    def start_scatter(c, b):
        pltpu.async_copy(rows.at[b], acc_sh.at[idx_d.at[c]], ssems[b],
                         add=True)

    def wait_scatter(c, b):
        pltpu.make_async_copy(rows.at[b], acc_sh.at[idx_d.at[c]],
                              ssems[b]).wait()

    for p in range(4):
        start_gather(p, p)

    def body(q, carry):
        for bb in range(8):
            c = 8 * q + bb
            wait_gather(c, bb)
            start_scatter(c, bb)

            @pl.when(c // (NCHUNK // 2) == cid)
            def _():
                pltpu.async_copy(ones_v, deg_sh.at[idx_d.at[c]], dsem,
                                 add=True)

            nb = (bb + 4) % 8

            @pl.when(c + 4 < NCHUNK)
            def _():
                @pl.when(c >= 4)
                def _():
                    wait_scatter(c - 4, nb)

                start_gather(c + 4, nb)

        return carry

    lax.fori_loop(0, NCHUNK // 8, body, 0)
    for bb in range(8):
        wait_scatter(NCHUNK - 8 + bb, bb)

    def drain_deg(c, carry):
        pltpu.make_async_copy(ones_v, deg_sh.at[idx_d.at[c]], dsem).wait()
        return carry

    half = NCHUNK // 2
    lax.fori_loop(cid * half, cid * half + half, drain_deg, 0)
    plsc.subcore_barrier()

    pltpu.sync_copy(acc_sh.at[pl.ds(base, ROWS_PER_TILE)],
                    acc_out.at[cid, pl.ds(base, ROWS_PER_TILE)])
    pltpu.sync_copy(deg_sh.at[pl.ds(base, ROWS_PER_TILE)],
                    deg_out.at[cid, pl.ds(base, ROWS_PER_TILE)])


ROW_BLK = 1000


def _tc_linear_body(h_ref, a0_ref, a1_ref, d0_ref, d1_ref,
                    w1_ref, w2a_ref, w2b_ref, b_ref, o_ref):
    deg = d0_ref[0, :, 0:1] + d1_ref[0, :, 0:1]
    rdeg = 1.0 / jnp.maximum(deg, 1.0)
    o_ref[...] = (
        jnp.dot(h_ref[...], w1_ref[...], preferred_element_type=jnp.float32)
        + jnp.dot(a0_ref[0] * rdeg, w2a_ref[...],
                  preferred_element_type=jnp.float32)
        + jnp.dot(a1_ref[0] * rdeg, w2b_ref[...],
                  preferred_element_type=jnp.float32)
        + b_ref[...]
    )


def _tc_linear(h, acc, deg, w1t, w2ta, w2tb, b2d):
    grid = (N_NODES // ROW_BLK,)
    return pl.pallas_call(
        _tc_linear_body,
        grid=grid,
        in_specs=[
            pl.BlockSpec((ROW_BLK, D_IN), lambda i: (i, 0)),
            pl.BlockSpec((1, ROW_BLK, D_HALF), lambda i: (0, i, 0)),
            pl.BlockSpec((1, ROW_BLK, D_HALF), lambda i: (1, i, 0)),
            pl.BlockSpec((1, ROW_BLK, DEG_W), lambda i: (0, i, 0)),
            pl.BlockSpec((1, ROW_BLK, DEG_W), lambda i: (1, i, 0)),
            pl.BlockSpec((D_IN, D_OUT), lambda i: (0, 0)),
            pl.BlockSpec((D_HALF, D_OUT), lambda i: (0, 0)),
            pl.BlockSpec((D_HALF, D_OUT), lambda i: (0, 0)),
            pl.BlockSpec((1, D_OUT), lambda i: (0, 0)),
        ],
        out_specs=pl.BlockSpec((ROW_BLK, D_OUT), lambda i: (i, 0)),
        out_shape=jax.ShapeDtypeStruct((N_NODES, D_OUT), jnp.float32),
    )(h, acc, acc, deg, deg, w1t, w2ta, w2tb, b2d)


def kernel(h, edge_index, W, b):
    src2 = (edge_index[0].astype(jnp.int32) * 2).reshape(NS, NCHUNK, CHUNK)
    dst = edge_index[1].astype(jnp.int32).reshape(NS, NCHUNK, CHUNK)
    h2 = h.astype(jnp.bfloat16).reshape(2 * N_NODES, D_HALF)
    ones = jnp.ones((CHUNK, DEG_W), dtype=jnp.float32)
    zacc = jnp.zeros((ROWS_PER_TILE, D_HALF), dtype=jnp.bfloat16)
    zdeg = jnp.zeros((ROWS_PER_TILE, DEG_W), dtype=jnp.float32)

    acc, deg = _sc_aggregate(h2, src2, dst, ones, zacc, zdeg)

    w1t = W[:, :D_IN].T
    w2ta = W[:, D_IN:D_IN + D_HALF].T
    w2tb = W[:, D_IN + D_HALF:].T
    b2d = b.reshape(1, D_OUT)
    return _tc_linear(h, acc, deg, w1t, w2ta, w2tb, b2d)


# speedup vs baseline: 15.9305x; 1.0054x over previous
import functools

import jax
import jax.numpy as jnp
from jax import lax
from jax.experimental import pallas as pl
from jax.experimental.pallas import tpu as pltpu
from jax.experimental.pallas import tpu_sc as plsc

N_NODES = 10000
N_EDGES = 320000
D_IN = 128
D_OUT = 128

NC = 2
NS = 16
D_HALF = D_IN // NC
E_PER_T = N_EDGES // NS
CHUNK = 125
NCHUNK = E_PER_T // CHUNK
N_PAD = 10240
ROWS_PER_TILE = N_PAD // NS
DEG_W = 16


@functools.partial(
    pl.kernel,
    out_type=(
        jax.ShapeDtypeStruct((NC, N_PAD, D_HALF), jnp.bfloat16),
        jax.ShapeDtypeStruct((NC, N_PAD, DEG_W), jnp.float32),
    ),
    mesh=plsc.VectorSubcoreMesh(core_axis_name="c", subcore_axis_name="s"),
    compiler_params=pltpu.CompilerParams(use_tc_tiling_on_sc=False),
    scratch_types=[
        pltpu.VMEM((NCHUNK, CHUNK), jnp.int32),
        pltpu.VMEM((NCHUNK, CHUNK), jnp.int32),
        pltpu.VMEM((8, CHUNK, D_HALF), jnp.bfloat16),
        pltpu.VMEM((CHUNK, DEG_W), jnp.float32),
        pltpu.VMEM_SHARED((N_PAD, D_HALF), jnp.bfloat16),
        pltpu.VMEM_SHARED((N_PAD, DEG_W), jnp.float32),
        [pltpu.SemaphoreType.DMA] * 8,
        [pltpu.SemaphoreType.DMA] * 8,
        pltpu.SemaphoreType.DMA,
    ],
)
def _sc_aggregate(h2_hbm, edges_hbm, ones_hbm, zacc_hbm, zdeg_hbm,
                  acc_out, deg_out,
                  idx_s, idx_d, rows, ones_v, acc_sh, deg_sh, gsems, ssems, dsem):
    cid = lax.axis_index("c")
    sid = lax.axis_index("s")

    pltpu.sync_copy(edges_hbm.at[0, sid], idx_s)
    pltpu.sync_copy(edges_hbm.at[1, sid], idx_d)
    pltpu.sync_copy(ones_hbm, ones_v)

    base = sid * ROWS_PER_TILE
    pltpu.sync_copy(zacc_hbm, acc_sh.at[pl.ds(base, ROWS_PER_TILE)])
    pltpu.sync_copy(zdeg_hbm, deg_sh.at[pl.ds(base, ROWS_PER_TILE)])
    plsc.subcore_barrier()

    h_view = h2_hbm.at[pl.ds(cid, 2 * N_NODES - 1)]

    def start_gather(c, b):
        pltpu.async_copy(h_view.at[idx_s.at[c]], rows.at[b], gsems[b])

    def wait_gather(c, b):
        pltpu.make_async_copy(h_view.at[idx_s.at[c]],
                              rows.at[b], gsems[b]).wait()

    def start_scatter(c, b):
        pltpu.async_copy(rows.at[b], acc_sh.at[idx_d.at[c]], ssems[b],
                         add=True)

    def wait_scatter(c, b):
        pltpu.make_async_copy(rows.at[b], acc_sh.at[idx_d.at[c]],
                              ssems[b]).wait()

    for p in range(4):
        start_gather(p, p)

    def body(q, carry):
        for bb in range(8):
            c = 8 * q + bb
            wait_gather(c, bb)
            start_scatter(c, bb)

            @pl.when(c // (NCHUNK // 2) == cid)
            def _():
                pltpu.async_copy(ones_v, deg_sh.at[idx_d.at[c]], dsem,
                                 add=True)

            nb = (bb + 4) % 8

            @pl.when(c + 4 < NCHUNK)
            def _():
                @pl.when(c >= 4)
                def _():
                    wait_scatter(c - 4, nb)

                start_gather(c + 4, nb)

        return carry

    lax.fori_loop(0, NCHUNK // 8, body, 0)
    for bb in range(8):
        wait_scatter(NCHUNK - 8 + bb, bb)

    def drain_deg(c, carry):
        pltpu.make_async_copy(ones_v, deg_sh.at[idx_d.at[c]], dsem).wait()
        return carry

    half = NCHUNK // 2
    lax.fori_loop(cid * half, cid * half + half, drain_deg, 0)
    plsc.subcore_barrier()

    pltpu.sync_copy(acc_sh.at[pl.ds(base, ROWS_PER_TILE)],
                    acc_out.at[cid, pl.ds(base, ROWS_PER_TILE)])
    pltpu.sync_copy(deg_sh.at[pl.ds(base, ROWS_PER_TILE)],
                    deg_out.at[cid, pl.ds(base, ROWS_PER_TILE)])


ROW_BLK = 1000


def _tc_self_body(h_ref, w1_ref, b_ref, o_ref):
    o_ref[...] = (
        jnp.dot(h_ref[...], w1_ref[...], preferred_element_type=jnp.float32)
        + b_ref[...]
    )


def _tc_neigh_body(p_ref, a0_ref, a1_ref, d0_ref, d1_ref,
                   w2a_ref, w2b_ref, o_ref):
    deg = d0_ref[0, :, 0:1] + d1_ref[0, :, 0:1]
    rdeg = 1.0 / jnp.maximum(deg, 1.0)
    o_ref[...] = (
        p_ref[...]
        + jnp.dot(a0_ref[0] * rdeg, w2a_ref[...],
                  preferred_element_type=jnp.float32)
        + jnp.dot(a1_ref[0] * rdeg, w2b_ref[...],
                  preferred_element_type=jnp.float32)
    )


def _tc_self(h, w1t, b2d):
    grid = (N_NODES // ROW_BLK,)
    return pl.pallas_call(
        _tc_self_body,
        grid=grid,
        in_specs=[
            pl.BlockSpec((ROW_BLK, D_IN), lambda i: (i, 0)),
            pl.BlockSpec((D_IN, D_OUT), lambda i: (0, 0)),
            pl.BlockSpec((1, D_OUT), lambda i: (0, 0)),
        ],
        out_specs=pl.BlockSpec((ROW_BLK, D_OUT), lambda i: (i, 0)),
        out_shape=jax.ShapeDtypeStruct((N_NODES, D_OUT), jnp.float32),
    )(h, w1t, b2d)


def _tc_neigh(partial, acc, deg, w2ta, w2tb):
    grid = (N_NODES // ROW_BLK,)
    return pl.pallas_call(
        _tc_neigh_body,
        grid=grid,
        in_specs=[
            pl.BlockSpec((ROW_BLK, D_OUT), lambda i: (i, 0)),
            pl.BlockSpec((1, ROW_BLK, D_HALF), lambda i: (0, i, 0)),
            pl.BlockSpec((1, ROW_BLK, D_HALF), lambda i: (1, i, 0)),
            pl.BlockSpec((1, ROW_BLK, DEG_W), lambda i: (0, i, 0)),
            pl.BlockSpec((1, ROW_BLK, DEG_W), lambda i: (1, i, 0)),
            pl.BlockSpec((D_HALF, D_OUT), lambda i: (0, 0)),
            pl.BlockSpec((D_HALF, D_OUT), lambda i: (0, 0)),
        ],
        out_specs=pl.BlockSpec((ROW_BLK, D_OUT), lambda i: (i, 0)),
        out_shape=jax.ShapeDtypeStruct((N_NODES, D_OUT), jnp.float32),
    )(partial, acc, acc, deg, deg, w2ta, w2tb)


def kernel(h, edge_index, W, b):
    edges = (edge_index.astype(jnp.int32)
             * jnp.array([[2], [1]], dtype=jnp.int32)
             ).reshape(2, NS, NCHUNK, CHUNK)
    h2 = h.astype(jnp.bfloat16).reshape(2 * N_NODES, D_HALF)
    ones = jnp.ones((CHUNK, DEG_W), dtype=jnp.float32)
    zacc = jnp.zeros((ROWS_PER_TILE, D_HALF), dtype=jnp.bfloat16)
    zdeg = jnp.zeros((ROWS_PER_TILE, DEG_W), dtype=jnp.float32)

    acc, deg = _sc_aggregate(h2, edges, ones, zacc, zdeg)

    w1t = W[:, :D_IN].T
    w2ta = W[:, D_IN:D_IN + D_HALF].T
    w2tb = W[:, D_IN + D_HALF:].T
    b2d = b.reshape(1, D_OUT)
    partial = _tc_self(h, w1t, b2d)
    return _tc_neigh(partial, acc, deg, w2ta, w2tb)


# speedup vs baseline: 16.2426x; 1.0196x over previous
import functools

import jax
import jax.numpy as jnp
from jax import lax
from jax.experimental import pallas as pl
from jax.experimental.pallas import tpu as pltpu
from jax.experimental.pallas import tpu_sc as plsc

N_NODES = 10000
N_EDGES = 320000
D_IN = 128
D_OUT = 128

NC = 2
NS = 16
NW = NC * NS
E_PER_W = N_EDGES // NW
CHUNK = 125
NCHUNK = E_PER_W // CHUNK
N_PAD = 10112
ROWS_PER_TILE = N_PAD // NS
DEG_W = 16
NBUF = 8
PF = 4


@functools.partial(
    pl.kernel,
    out_type=(
        jax.ShapeDtypeStruct((NC, N_PAD, D_IN), jnp.bfloat16),
        jax.ShapeDtypeStruct((NC, N_PAD, DEG_W), jnp.bfloat16),
    ),
    mesh=plsc.VectorSubcoreMesh(core_axis_name="c", subcore_axis_name="s"),
    compiler_params=pltpu.CompilerParams(use_tc_tiling_on_sc=False),
    scratch_types=[
        pltpu.VMEM((NCHUNK, CHUNK), jnp.int32),
        pltpu.VMEM((NCHUNK, CHUNK), jnp.int32),
        pltpu.VMEM((NBUF, CHUNK, D_IN), jnp.bfloat16),
        pltpu.VMEM((CHUNK, DEG_W), jnp.bfloat16),
        pltpu.VMEM_SHARED((N_PAD, D_IN), jnp.bfloat16),
        pltpu.VMEM_SHARED((N_PAD, DEG_W), jnp.bfloat16),
        [pltpu.SemaphoreType.DMA] * NBUF,
        [pltpu.SemaphoreType.DMA] * NBUF,
        pltpu.SemaphoreType.DMA,
    ],
)
def _sc_aggregate(hb_hbm, edges_hbm, ones_hbm, zacc_hbm, zdeg_hbm,
                  acc_out, deg_out,
                  idx_s, idx_d, rows, ones_v, acc_sh, deg_sh,
                  gsems, ssems, dsem):
    cid = lax.axis_index("c")
    sid = lax.axis_index("s")
    wid = sid * NC + cid

    pltpu.sync_copy(edges_hbm.at[0, wid], idx_s)
    pltpu.sync_copy(edges_hbm.at[1, wid], idx_d)
    pltpu.sync_copy(ones_hbm, ones_v)

    base = sid * ROWS_PER_TILE
    pltpu.sync_copy(zacc_hbm, acc_sh.at[pl.ds(base, ROWS_PER_TILE)])
    pltpu.sync_copy(zdeg_hbm, deg_sh.at[pl.ds(base, ROWS_PER_TILE)])
    plsc.subcore_barrier()

    def start_gather(c, b):
        pltpu.async_copy(hb_hbm.at[idx_s.at[c]], rows.at[b], gsems[b])

    def wait_gather(c, b):
        pltpu.make_async_copy(hb_hbm.at[idx_s.at[c]],
                              rows.at[b], gsems[b]).wait()

    def start_scatter(c, b):
        pltpu.async_copy(rows.at[b], acc_sh.at[idx_d.at[c]], ssems[b],
                         add=True)

    def wait_scatter(c, b):
        pltpu.make_async_copy(rows.at[b], acc_sh.at[idx_d.at[c]],
                              ssems[b]).wait()

    for p in range(PF):
        start_gather(p, p)

    def body(q, carry):
        for bb in range(NBUF):
            c = NBUF * q + bb
            wait_gather(c, bb)
            start_scatter(c, bb)

            pltpu.async_copy(ones_v, deg_sh.at[idx_d.at[c]], dsem, add=True)

            nb = (bb + PF) % NBUF

            @pl.when(c + PF < NCHUNK)
            def _():
                @pl.when(c >= PF)
                def _():
                    wait_scatter(c - PF, nb)

                start_gather(c + PF, nb)

        return carry

    lax.fori_loop(0, NCHUNK // NBUF, body, 0)
    for bb in range(NBUF):
        wait_scatter(NCHUNK - NBUF + bb, bb)

    def drain_deg(c, carry):
        pltpu.make_async_copy(ones_v, deg_sh.at[idx_d.at[c]], dsem).wait()
        return carry

    lax.fori_loop(0, NCHUNK, drain_deg, 0)
    plsc.subcore_barrier()

    pltpu.sync_copy(acc_sh.at[pl.ds(base, ROWS_PER_TILE)],
                    acc_out.at[cid, pl.ds(base, ROWS_PER_TILE)])
    pltpu.sync_copy(deg_sh.at[pl.ds(base, ROWS_PER_TILE)],
                    deg_out.at[cid, pl.ds(base, ROWS_PER_TILE)])


ROW_BLK = 2000


def _tc_self_body(h_ref, w1_ref, b_ref, o_ref):
    o_ref[...] = (
        jnp.dot(h_ref[...], w1_ref[...], preferred_element_type=jnp.float32)
        + b_ref[...]
    )


def _tc_neigh_body(p_ref, a0_ref, a1_ref, d0_ref, d1_ref, w2_ref, o_ref):
    deg = (d0_ref[0, :, 0:1].astype(jnp.float32)
           + d1_ref[0, :, 0:1].astype(jnp.float32))
    rdeg = 1.0 / jnp.maximum(deg, 1.0)
    h_n = (a0_ref[0].astype(jnp.float32) + a1_ref[0].astype(jnp.float32))
    o_ref[...] = (
        p_ref[...]
        + jnp.dot(h_n * rdeg, w2_ref[...], preferred_element_type=jnp.float32)
    )


def _tc_self(h, w1t, b2d):
    grid = (N_NODES // ROW_BLK,)
    return pl.pallas_call(
        _tc_self_body,
        grid=grid,
        in_specs=[
            pl.BlockSpec((ROW_BLK, D_IN), lambda i: (i, 0)),
            pl.BlockSpec((D_IN, D_OUT), lambda i: (0, 0)),
            pl.BlockSpec((1, D_OUT), lambda i: (0, 0)),
        ],
        out_specs=pl.BlockSpec((ROW_BLK, D_OUT), lambda i: (i, 0)),
        out_shape=jax.ShapeDtypeStruct((N_NODES, D_OUT), jnp.float32),
    )(h, w1t, b2d)


def _tc_neigh(partial, acc, deg, w2t):
    grid = (N_NODES // ROW_BLK,)
    return pl.pallas_call(
        _tc_neigh_body,
        grid=grid,
        in_specs=[
            pl.BlockSpec((ROW_BLK, D_OUT), lambda i: (i, 0)),
            pl.BlockSpec((1, ROW_BLK, D_IN), lambda i: (0, i, 0)),
            pl.BlockSpec((1, ROW_BLK, D_IN), lambda i: (1, i, 0)),
            pl.BlockSpec((1, ROW_BLK, DEG_W), lambda i: (0, i, 0)),
            pl.BlockSpec((1, ROW_BLK, DEG_W), lambda i: (1, i, 0)),
            pl.BlockSpec((D_IN, D_OUT), lambda i: (0, 0)),
        ],
        out_specs=pl.BlockSpec((ROW_BLK, D_OUT), lambda i: (i, 0)),
        out_shape=jax.ShapeDtypeStruct((N_NODES, D_OUT), jnp.float32),
    )(partial, acc, acc, deg, deg, w2t)


def kernel(h, edge_index, W, b):
    edges = edge_index.astype(jnp.int32).reshape(2, NW, NCHUNK, CHUNK)
    hb = h.astype(jnp.bfloat16)
    ones = jnp.ones((CHUNK, DEG_W), dtype=jnp.bfloat16)
    zacc = jnp.zeros((ROWS_PER_TILE, D_IN), dtype=jnp.bfloat16)
    zdeg = jnp.zeros((ROWS_PER_TILE, DEG_W), dtype=jnp.bfloat16)

    acc, deg = _sc_aggregate(hb, edges, ones, zacc, zdeg)

    w1t = W[:, :D_IN].T
    w2t = W[:, D_IN:].T
    b2d = b.reshape(1, D_OUT)
    partial = _tc_self(h, w1t, b2d)
    return _tc_neigh(partial, acc, deg, w2t)
